# baseline (speedup 1.0000x reference)
import jax
import jax.numpy as jnp
from jax import lax
from jax.experimental import pallas as pl
from jax.experimental.pallas import tpu as pltpu

_N = 128
_D = 6
_ND = 7
_BF = 13
_AF = 128
_H = 1024


def _mol_kernel(atoms_ref, bonds_ref, edges_ref,
                w1a_ref, w1b_ref, b1_ref,
                w2a_ref, w2b_ref, b2_ref,
                woa_ref, wob_ref, bo_ref,
                out_ref):
    f32 = jnp.float32
    bf16 = jnp.bfloat16
    x = atoms_ref[0]
    b78 = bonds_ref[0]
    e = edges_ref[0]

    si = lax.broadcasted_iota(jnp.int32, (_D * _BF, _BF), 0)
    sj = lax.broadcasted_iota(jnp.int32, (_D * _BF, _BF), 1)
    sel = (si % _BF == sj).astype(f32)
    sb = jnp.dot(b78, sel, preferred_element_type=f32)
    sbb = sb.astype(bf16)

    colids = lax.broadcasted_iota(jnp.int32, (_N, _N), 1)
    rowids = lax.broadcasted_iota(jnp.int32, (_N, _N), 0)
    eye = (colids == rowids).astype(f32)

    onehots = [(e[:, d:d + 1] == colids).astype(bf16) for d in range(_D)]
    A = eye.astype(bf16)
    for oh in onehots:
        A = A + oh
    vbias = [jnp.where(e[:, d:d + 1] >= 0, 0.0, -1e30).astype(f32)
             for d in range(_D)]

    deg = jnp.sum((e != -1).astype(f32), axis=1, keepdims=True)

    def conv(xin, wa, wbflat, bflat):
        s_atoms = jnp.dot(A, xin.astype(bf16),
                          preferred_element_type=f32)
        z_all = (jnp.dot(s_atoms.astype(bf16), wa, preferred_element_type=f32)
                 + jnp.dot(sbb, wbflat, preferred_element_type=f32)
                 + bflat)
        zsel = z_all[:, 0:128]
        for d in range(1, _ND):
            zsel = jnp.where(deg == d, z_all[:, d * 128:(d + 1) * 128], zsel)
        return jnp.maximum(zsel, 0.0)

    def pool(h):
        g = h
        hb = h.astype(bf16)
        for d in range(_D):
            gd = jnp.dot(onehots[d], hb, preferred_element_type=f32)
            g = jnp.maximum(g, gd + vbias[d])
        return g

    h1 = conv(x, w1a_ref[...], w1b_ref[...], b1_ref[...])
    p1 = pool(h1)
    h2 = conv(p1, w2a_ref[...], w2b_ref[...], b2_ref[...])
    p2 = pool(h2)

    z = (jnp.dot(p2.astype(bf16), woa_ref[...], preferred_element_type=f32)
         + jnp.dot(sbb, wob_ref[...], preferred_element_type=f32)
         + bo_ref[...])
    mrow = jnp.swapaxes((deg != 0).astype(f32), 0, 1)
    out_ref[0] = jnp.dot(mrow, jnp.tanh(z), preferred_element_type=f32)


def kernel(atoms, bonds, edges, W1, b1, W2, b2, Wo, bo):
    B = atoms.shape[0]
    b78 = bonds.reshape(B, _N, _D * _BF)

    def split_w(W, b):
        wa = jnp.transpose(W[:, :_AF, :], (1, 0, 2)).reshape(_AF, _ND * 128)
        wb = jnp.transpose(W[:, _AF:, :], (1, 0, 2)).reshape(_BF, _ND * 128)
        return wa.astype(jnp.bfloat16), wb.astype(jnp.bfloat16), \
            b.reshape(1, _ND * 128)

    w1a, w1b, b1f = split_w(W1, b1)
    w2a, w2b, b2f = split_w(W2, b2)
    woa = Wo[:_AF].astype(jnp.bfloat16)
    wob = Wo[_AF:].astype(jnp.bfloat16)
    bof = bo.reshape(1, _H)

    const = lambda i: (0, 0)
    return pl.pallas_call(
        _mol_kernel,
        grid=(B,),
        in_specs=[
            pl.BlockSpec((1, _N, _AF), lambda i: (i, 0, 0)),
            pl.BlockSpec((1, _N, _D * _BF), lambda i: (i, 0, 0)),
            pl.BlockSpec((1, _N, _D), lambda i: (i, 0, 0)),
            pl.BlockSpec((_AF, _ND * 128), const),
            pl.BlockSpec((_BF, _ND * 128), const),
            pl.BlockSpec((1, _ND * 128), const),
            pl.BlockSpec((_AF, _ND * 128), const),
            pl.BlockSpec((_BF, _ND * 128), const),
            pl.BlockSpec((1, _ND * 128), const),
            pl.BlockSpec((_AF, _H), const),
            pl.BlockSpec((_BF, _H), const),
            pl.BlockSpec((1, _H), const),
        ],
        out_specs=pl.BlockSpec((1, 1, _H), lambda i: (i, 0, 0)),
        out_shape=jax.ShapeDtypeStruct((B, 1, _H), jnp.float32),
        compiler_params=pltpu.CompilerParams(
            dimension_semantics=("parallel",)),
    )(atoms, b78, edges, w1a, w1b, b1f, w2a, w2b, b2f, woa, wob, bof
      ).reshape(B, _H)


# speedup vs baseline: 1.0621x; 1.0621x over previous
import jax
import jax.numpy as jnp
from jax import lax
from jax.experimental import pallas as pl
from jax.experimental.pallas import tpu as pltpu

_N = 128
_D = 6
_ND = 7
_BF = 13
_AF = 128
_H = 1024
_G = 2


def _mol_kernel(atoms_ref, bonds_ref, edges_ref,
                w1a_ref, w1b_ref, b1_ref,
                w2a_ref, w2b_ref, b2_ref,
                woa_ref, wob_ref, bo_ref,
                out_ref):
    f32 = jnp.float32
    bf16 = jnp.bfloat16

    si = lax.broadcasted_iota(jnp.int32, (_D * _BF, _BF), 0)
    sj = lax.broadcasted_iota(jnp.int32, (_D * _BF, _BF), 1)
    sel = (si % _BF == sj).astype(f32)
    colids = lax.broadcasted_iota(jnp.int32, (_N, _N), 1)
    rowids = lax.broadcasted_iota(jnp.int32, (_N, _N), 0)
    eye_b = (colids == rowids).astype(bf16)

    def run_mol(m):
        x = atoms_ref[m]
        b78 = bonds_ref[m]
        e = edges_ref[m]

        sb = jnp.dot(b78, sel, preferred_element_type=f32)
        sbb = sb.astype(bf16)

        onehots = [(e[:, d:d + 1] == colids).astype(bf16) for d in range(_D)]
        A = eye_b
        for oh in onehots:
            A = A + oh
        vbias = [jnp.where(e[:, d:d + 1] >= 0, 0.0, -1e30).astype(f32)
                 for d in range(_D)]

        deg = jnp.sum((e != -1).astype(f32), axis=1, keepdims=True)

        def conv(xin, wa, wbflat, bflat):
            s_atoms = jnp.dot(A, xin.astype(bf16),
                              preferred_element_type=f32)
            z_all = (jnp.dot(s_atoms.astype(bf16), wa,
                             preferred_element_type=f32)
                     + jnp.dot(sbb, wbflat, preferred_element_type=f32)
                     + bflat)
            zsel = z_all[:, 0:128]
            for d in range(1, _ND):
                zsel = jnp.where(deg == d,
                                 z_all[:, d * 128:(d + 1) * 128], zsel)
            return jnp.maximum(zsel, 0.0)

        def pool(h):
            g = h
            hb = h.astype(bf16)
            for d in range(_D):
                gd = jnp.dot(onehots[d], hb, preferred_element_type=f32)
                g = jnp.maximum(g, gd + vbias[d])
            return g

        h1 = conv(x, w1a_ref[...], w1b_ref[...], b1_ref[...])
        p1 = pool(h1)
        h2 = conv(p1, w2a_ref[...], w2b_ref[...], b2_ref[...])
        p2 = pool(h2)

        z = (jnp.dot(p2.astype(bf16), woa_ref[...],
                     preferred_element_type=f32)
             + jnp.dot(sbb, wob_ref[...], preferred_element_type=f32)
             + bo_ref[...])
        mrow = jnp.swapaxes((deg != 0).astype(f32), 0, 1)
        return jnp.dot(mrow, jnp.tanh(z), preferred_element_type=f32)

    for m in range(_G):
        out_ref[m] = run_mol(m)


def kernel(atoms, bonds, edges, W1, b1, W2, b2, Wo, bo):
    B = atoms.shape[0]
    b78 = bonds.reshape(B, _N, _D * _BF)

    def split_w(W, b):
        wa = jnp.transpose(W[:, :_AF, :], (1, 0, 2)).reshape(_AF, _ND * 128)
        wb = jnp.transpose(W[:, _AF:, :], (1, 0, 2)).reshape(_BF, _ND * 128)
        return wa.astype(jnp.bfloat16), wb.astype(jnp.bfloat16), \
            b.reshape(1, _ND * 128)

    w1a, w1b, b1f = split_w(W1, b1)
    w2a, w2b, b2f = split_w(W2, b2)
    woa = Wo[:_AF].astype(jnp.bfloat16)
    wob = Wo[_AF:].astype(jnp.bfloat16)
    bof = bo.reshape(1, _H)

    const = lambda i: (0, 0)
    return pl.pallas_call(
        _mol_kernel,
        grid=(B // _G,),
        in_specs=[
            pl.BlockSpec((_G, _N, _AF), lambda i: (i, 0, 0)),
            pl.BlockSpec((_G, _N, _D * _BF), lambda i: (i, 0, 0)),
            pl.BlockSpec((_G, _N, _D), lambda i: (i, 0, 0)),
            pl.BlockSpec((_AF, _ND * 128), const),
            pl.BlockSpec((_BF, _ND * 128), const),
            pl.BlockSpec((1, _ND * 128), const),
            pl.BlockSpec((_AF, _ND * 128), const),
            pl.BlockSpec((_BF, _ND * 128), const),
            pl.BlockSpec((1, _ND * 128), const),
            pl.BlockSpec((_AF, _H), const),
            pl.BlockSpec((_BF, _H), const),
            pl.BlockSpec((1, _H), const),
        ],
        out_specs=pl.BlockSpec((_G, 1, _H), lambda i: (i, 0, 0)),
        out_shape=jax.ShapeDtypeStruct((B, 1, _H), jnp.float32),
        compiler_params=pltpu.CompilerParams(
            dimension_semantics=("parallel",)),
    )(atoms, b78, edges, w1a, w1b, b1f, w2a, w2b, b2f, woa, wob, bof
      ).reshape(B, _H)


# speedup vs baseline: 1.1014x; 1.0370x over previous
import jax
import jax.numpy as jnp
from jax import lax
from jax.experimental import pallas as pl
from jax.experimental.pallas import tpu as pltpu

_N = 128
_D = 6
_ND = 7
_BF = 13
_AF = 128
_H = 1024
_G = 2


def _mol_kernel(atoms_ref, bonds_ref, edges_ref,
                w1_ref, w2_ref, wo_ref,
                out_ref):
    f32 = jnp.float32
    bf16 = jnp.bfloat16

    si = lax.broadcasted_iota(jnp.int32, (_D * _BF, _BF), 0)
    sj = lax.broadcasted_iota(jnp.int32, (_D * _BF, _BF), 1)
    sel = (si % _BF == sj).astype(f32)
    colids = lax.broadcasted_iota(jnp.int32, (_N, _N), 1)
    rowids = lax.broadcasted_iota(jnp.int32, (_N, _N), 0)
    eye_b = (colids == rowids).astype(bf16)

    def run_mol(m):
        x = atoms_ref[m]
        b78 = bonds_ref[m]
        e = edges_ref[m]

        sb = jnp.dot(b78, sel, preferred_element_type=f32)
        sb_ext = jnp.concatenate(
            [sb.astype(bf16),
             jnp.ones((_N, 1), bf16),
             jnp.zeros((_N, _AF - _BF - 1), bf16)], axis=1)

        onehots = [(e[:, d:d + 1] == colids).astype(bf16) for d in range(_D)]
        A = eye_b
        for oh in onehots:
            A = A + oh
        vbias = [jnp.where(e[:, d:d + 1] >= 0, 0.0, -1e30).astype(f32)
                 for d in range(_D)]

        deg = jnp.sum((e != -1).astype(f32), axis=1, keepdims=True)

        def conv(xin, wcat):
            s_atoms = jnp.dot(A, xin.astype(bf16),
                              preferred_element_type=f32)
            lhs = jnp.concatenate([s_atoms.astype(bf16), sb_ext], axis=1)
            z_all = jnp.dot(lhs, wcat,
                            preferred_element_type=f32)
            zsel = z_all[:, 0:128]
            for d in range(1, _ND):
                zsel = jnp.where(deg == d,
                                 z_all[:, d * 128:(d + 1) * 128], zsel)
            return jnp.maximum(zsel, 0.0)

        def pool(h):
            g = h
            hb = h.astype(bf16)
            for d in range(_D):
                gd = jnp.dot(onehots[d], hb, preferred_element_type=f32)
                g = jnp.maximum(g, gd + vbias[d])
            return g

        h1 = conv(x, w1_ref[...])
        p1 = pool(h1)
        h2 = conv(p1, w2_ref[...])
        p2 = pool(h2)

        lhs = jnp.concatenate([p2.astype(bf16), sb_ext], axis=1)
        z = jnp.dot(lhs, wo_ref[...], preferred_element_type=f32)
        mrow = jnp.swapaxes((deg != 0).astype(f32), 0, 1)
        return jnp.dot(mrow, jnp.tanh(z), preferred_element_type=f32)

    for m in range(_G):
        out_ref[m] = run_mol(m)


def kernel(atoms, bonds, edges, W1, b1, W2, b2, Wo, bo):
    B = atoms.shape[0]
    b78 = bonds.reshape(B, _N, _D * _BF)

    def fuse_w(W, b, nout):
        wa = jnp.transpose(W[:, :_AF, :], (1, 0, 2)).reshape(_AF, nout)
        wb = jnp.transpose(W[:, _AF:, :], (1, 0, 2)).reshape(_BF, nout)
        return jnp.concatenate(
            [wa, wb, b.reshape(1, nout),
             jnp.zeros((_AF - _BF - 1, nout), W.dtype)],
            axis=0).astype(jnp.bfloat16)

    w1c = fuse_w(W1, b1, _ND * 128)
    w2c = fuse_w(W2, b2, _ND * 128)
    woc = jnp.concatenate(
        [Wo, bo.reshape(1, _H), jnp.zeros((_AF - _BF - 1, _H), Wo.dtype)],
        axis=0).astype(jnp.bfloat16)

    const = lambda i: (0, 0)
    return pl.pallas_call(
        _mol_kernel,
        grid=(B // _G,),
        in_specs=[
            pl.BlockSpec((_G, _N, _AF), lambda i: (i, 0, 0)),
            pl.BlockSpec((_G, _N, _D * _BF), lambda i: (i, 0, 0)),
            pl.BlockSpec((_G, _N, _D), lambda i: (i, 0, 0)),
            pl.BlockSpec((2 * _AF, _ND * 128), const),
            pl.BlockSpec((2 * _AF, _ND * 128), const),
            pl.BlockSpec((2 * _AF, _H), const),
        ],
        out_specs=pl.BlockSpec((_G, 1, _H), lambda i: (i, 0, 0)),
        out_shape=jax.ShapeDtypeStruct((B, 1, _H), jnp.float32),
        compiler_params=pltpu.CompilerParams(
            dimension_semantics=("parallel",)),
    )(atoms, b78, edges, w1c, w2c, woc).reshape(B, _H)


# speedup vs baseline: 1.1467x; 1.0411x over previous
import jax
import jax.numpy as jnp
from jax import lax
from jax.experimental import pallas as pl
from jax.experimental.pallas import tpu as pltpu

_N = 128
_D = 6
_ND = 7
_BF = 13
_AF = 128
_H = 1024
_G = 4


def _mol_kernel(atoms_ref, bonds_ref, edges_ref,
                w1_ref, w2_ref, wo_ref,
                out_ref):
    f32 = jnp.float32
    bf16 = jnp.bfloat16

    si = lax.broadcasted_iota(jnp.int32, (_D * _BF, _BF), 0)
    sj = lax.broadcasted_iota(jnp.int32, (_D * _BF, _BF), 1)
    sel = (si % _BF == sj).astype(f32)
    colids = lax.broadcasted_iota(jnp.int32, (_N, _N), 1)
    rowids = lax.broadcasted_iota(jnp.int32, (_N, _N), 0)
    eye_b = (colids == rowids).astype(bf16)

    def run_mol(m):
        x = atoms_ref[m]
        b78 = bonds_ref[m]
        e = edges_ref[m]

        sb = jnp.dot(b78, sel, preferred_element_type=f32)
        sb_ext = jnp.concatenate(
            [sb.astype(bf16),
             jnp.ones((_N, 1), bf16),
             jnp.zeros((_N, _AF - _BF - 1), bf16)], axis=1)

        onehots = [(e[:, d:d + 1] == colids).astype(bf16) for d in range(_D)]
        A = eye_b
        for oh in onehots:
            A = A + oh
        vbias = [jnp.where(e[:, d:d + 1] >= 0, 0.0, -1e30).astype(f32)
                 for d in range(_D)]

        deg = jnp.sum((e != -1).astype(f32), axis=1, keepdims=True)

        def conv(xin, wcat):
            s_atoms = jnp.dot(A, xin.astype(bf16),
                              preferred_element_type=f32)
            lhs = jnp.concatenate([s_atoms.astype(bf16), sb_ext], axis=1)
            z_all = jnp.dot(lhs, wcat,
                            preferred_element_type=f32)
            zsel = z_all[:, 0:128]
            for d in range(1, _ND):
                zsel = jnp.where(deg == d,
                                 z_all[:, d * 128:(d + 1) * 128], zsel)
            return jnp.maximum(zsel, 0.0)

        def pool(h):
            g = h
            hb = h.astype(bf16)
            for d in range(_D):
                gd = jnp.dot(onehots[d], hb, preferred_element_type=f32)
                g = jnp.maximum(g, gd + vbias[d])
            return g

        h1 = conv(x, w1_ref[...])
        p1 = pool(h1)
        h2 = conv(p1, w2_ref[...])
        p2 = pool(h2)

        lhs = jnp.concatenate([p2.astype(bf16), sb_ext], axis=1)
        z = jnp.dot(lhs, wo_ref[...], preferred_element_type=f32)
        mrow = jnp.swapaxes((deg != 0).astype(f32), 0, 1)
        return jnp.dot(mrow, jnp.tanh(z), preferred_element_type=f32)

    for m in range(_G):
        out_ref[m] = run_mol(m)


def kernel(atoms, bonds, edges, W1, b1, W2, b2, Wo, bo):
    B = atoms.shape[0]
    b78 = bonds.reshape(B, _N, _D * _BF)

    def fuse_w(W, b, nout):
        wa = jnp.transpose(W[:, :_AF, :], (1, 0, 2)).reshape(_AF, nout)
        wb = jnp.transpose(W[:, _AF:, :], (1, 0, 2)).reshape(_BF, nout)
        return jnp.concatenate(
            [wa, wb, b.reshape(1, nout),
             jnp.zeros((_AF - _BF - 1, nout), W.dtype)],
            axis=0).astype(jnp.bfloat16)

    w1c = fuse_w(W1, b1, _ND * 128)
    w2c = fuse_w(W2, b2, _ND * 128)
    woc = jnp.concatenate(
        [Wo, bo.reshape(1, _H), jnp.zeros((_AF - _BF - 1, _H), Wo.dtype)],
        axis=0).astype(jnp.bfloat16)

    const = lambda i: (0, 0)
    return pl.pallas_call(
        _mol_kernel,
        grid=(B // _G,),
        in_specs=[
            pl.BlockSpec((_G, _N, _AF), lambda i: (i, 0, 0)),
            pl.BlockSpec((_G, _N, _D * _BF), lambda i: (i, 0, 0)),
            pl.BlockSpec((_G, _N, _D), lambda i: (i, 0, 0)),
            pl.BlockSpec((2 * _AF, _ND * 128), const),
            pl.BlockSpec((2 * _AF, _ND * 128), const),
            pl.BlockSpec((2 * _AF, _H), const),
        ],
        out_specs=pl.BlockSpec((_G, 1, _H), lambda i: (i, 0, 0)),
        out_shape=jax.ShapeDtypeStruct((B, 1, _H), jnp.float32),
        compiler_params=pltpu.CompilerParams(
            dimension_semantics=("parallel",)),
    )(atoms, b78, edges, w1c, w2c, woc).reshape(B, _H)


# speedup vs baseline: 1.1676x; 1.0182x over previous
import jax
import jax.numpy as jnp
from jax import lax
from jax.experimental import pallas as pl
from jax.experimental.pallas import tpu as pltpu

_N = 128
_D = 6
_ND = 7
_BF = 13
_AF = 128
_H = 1024
_G = 8


def _mol_kernel(atoms_ref, bonds_ref, edges_ref,
                w1_ref, w2_ref, wo_ref,
                out_ref):
    f32 = jnp.float32
    bf16 = jnp.bfloat16

    si = lax.broadcasted_iota(jnp.int32, (_D * _BF, _BF), 0)
    sj = lax.broadcasted_iota(jnp.int32, (_D * _BF, _BF), 1)
    sel = (si % _BF == sj).astype(f32)
    colids = lax.broadcasted_iota(jnp.int32, (_N, _N), 1)
    rowids = lax.broadcasted_iota(jnp.int32, (_N, _N), 0)
    eye_b = (colids == rowids).astype(bf16)

    def run_mol(m):
        x = atoms_ref[m]
        b78 = bonds_ref[m]
        e = edges_ref[m]

        sb = jnp.dot(b78, sel, preferred_element_type=f32)
        sb_ext = jnp.concatenate(
            [sb.astype(bf16),
             jnp.ones((_N, 1), bf16),
             jnp.zeros((_N, _AF - _BF - 1), bf16)], axis=1)

        onehots = [(e[:, d:d + 1] == colids).astype(bf16) for d in range(_D)]
        A = eye_b
        for oh in onehots:
            A = A + oh
        vbias = [jnp.where(e[:, d:d + 1] >= 0, 0.0, -1e30).astype(f32)
                 for d in range(_D)]

        deg = jnp.sum((e != -1).astype(f32), axis=1, keepdims=True)

        def conv(xin, wcat):
            s_atoms = jnp.dot(A, xin.astype(bf16),
                              preferred_element_type=f32)
            lhs = jnp.concatenate([s_atoms.astype(bf16), sb_ext], axis=1)
            z_all = jnp.dot(lhs, wcat,
                            preferred_element_type=f32)
            zsel = z_all[:, 0:128]
            for d in range(1, _ND):
                zsel = jnp.where(deg == d,
                                 z_all[:, d * 128:(d + 1) * 128], zsel)
            return jnp.maximum(zsel, 0.0)

        def pool(h):
            g = h
            hb = h.astype(bf16)
            for d in range(_D):
                gd = jnp.dot(onehots[d], hb, preferred_element_type=f32)
                g = jnp.maximum(g, gd + vbias[d])
            return g

        h1 = conv(x, w1_ref[...])
        p1 = pool(h1)
        h2 = conv(p1, w2_ref[...])
        p2 = pool(h2)

        lhs = jnp.concatenate([p2.astype(bf16), sb_ext], axis=1)
        z = jnp.dot(lhs, wo_ref[...], preferred_element_type=f32)
        mrow = jnp.swapaxes((deg != 0).astype(f32), 0, 1)
        return jnp.dot(mrow, jnp.tanh(z), preferred_element_type=f32)

    for m in range(_G):
        out_ref[m] = run_mol(m)


def kernel(atoms, bonds, edges, W1, b1, W2, b2, Wo, bo):
    B = atoms.shape[0]
    b78 = bonds.reshape(B, _N, _D * _BF)

    def fuse_w(W, b, nout):
        wa = jnp.transpose(W[:, :_AF, :], (1, 0, 2)).reshape(_AF, nout)
        wb = jnp.transpose(W[:, _AF:, :], (1, 0, 2)).reshape(_BF, nout)
        return jnp.concatenate(
            [wa, wb, b.reshape(1, nout),
             jnp.zeros((_AF - _BF - 1, nout), W.dtype)],
            axis=0).astype(jnp.bfloat16)

    w1c = fuse_w(W1, b1, _ND * 128)
    w2c = fuse_w(W2, b2, _ND * 128)
    woc = jnp.concatenate(
        [Wo, bo.reshape(1, _H), jnp.zeros((_AF - _BF - 1, _H), Wo.dtype)],
        axis=0).astype(jnp.bfloat16)

    const = lambda i: (0, 0)
    return pl.pallas_call(
        _mol_kernel,
        grid=(B // _G,),
        in_specs=[
            pl.BlockSpec((_G, _N, _AF), lambda i: (i, 0, 0)),
            pl.BlockSpec((_G, _N, _D * _BF), lambda i: (i, 0, 0)),
            pl.BlockSpec((_G, _N, _D), lambda i: (i, 0, 0)),
            pl.BlockSpec((2 * _AF, _ND * 128), const),
            pl.BlockSpec((2 * _AF, _ND * 128), const),
            pl.BlockSpec((2 * _AF, _H), const),
        ],
        out_specs=pl.BlockSpec((_G, 1, _H), lambda i: (i, 0, 0)),
        out_shape=jax.ShapeDtypeStruct((B, 1, _H), jnp.float32),
        compiler_params=pltpu.CompilerParams(
            dimension_semantics=("parallel",)),
    )(atoms, b78, edges, w1c, w2c, woc).reshape(B, _H)
